# baseline (speedup 1.0000x reference)
import math

import jax
import jax.numpy as jnp
from jax import lax
from jax.experimental import pallas as pl
from jax.experimental.pallas import tpu as pltpu
from jax.experimental.pallas import tpu_sc as plsc

D_MODEL = 64
VOCAB = 1000000
B_TOTAL = 4096 * 200
SCALE = math.sqrt(D_MODEL)

NC, NS, L = 2, 16, 16
NW = NC * NS
CHUNK = 128
NBUF = 8
ROWS_PER_W = B_TOTAL // NW
CHUNKS_PER_W = ROWS_PER_W // CHUNK
OUTER = CHUNKS_PER_W // NBUF


def _emb_body(x_hbm, table_hbm, out_hbm, idx_v, *scratch):
    rows = scratch[:NBUF]
    gsem = scratch[NBUF:2 * NBUF]
    ssem = scratch[2 * NBUF:3 * NBUF]

    wid = lax.axis_index("s") * NC + lax.axis_index("c")
    chunk0 = wid * CHUNKS_PER_W

    pltpu.sync_copy(x_hbm.at[pl.ds(chunk0, CHUNKS_PER_W)], idx_v)

    def outer(o, carry):
        for b in range(NBUF):
            g = o * NBUF + b

            @pl.when(o > 0)
            def _drain_prev_scatter(b=b):
                pltpu.make_async_copy(
                    rows[b], out_hbm.at[pl.ds(0, CHUNK)], ssem[b]).wait()

            pltpu.async_copy(table_hbm.at[idx_v.at[g]], rows[b], gsem[b])
        for b in range(NBUF):
            g = o * NBUF + b
            pltpu.make_async_copy(
                table_hbm.at[idx_v.at[g]], rows[b], gsem[b]).wait()

            def scale_row(i, c, b=b):
                for j in range(D_MODEL // L):
                    rows[b][i, pl.ds(j * L, L)] = (
                        rows[b][i, pl.ds(j * L, L)] * SCALE)
                return c

            lax.fori_loop(0, CHUNK, scale_row, 0)
            pltpu.async_copy(
                rows[b], out_hbm.at[pl.ds((chunk0 + g) * CHUNK, CHUNK)],
                ssem[b])
        return carry

    lax.fori_loop(0, OUTER, outer, 0)
    for b in range(NBUF):
        pltpu.make_async_copy(
            rows[b], out_hbm.at[pl.ds(0, CHUNK)], ssem[b]).wait()


_emb = pl.kernel(
    _emb_body,
    out_type=jax.ShapeDtypeStruct((B_TOTAL, D_MODEL), jnp.float32),
    mesh=plsc.VectorSubcoreMesh(
        core_axis_name="c", subcore_axis_name="s",
        num_cores=NC, num_subcores=NS),
    compiler_params=pltpu.CompilerParams(use_tc_tiling_on_sc=False),
    scratch_types=(
        [pltpu.VMEM((CHUNKS_PER_W, CHUNK), jnp.int32)]
        + [pltpu.VMEM((CHUNK, D_MODEL), jnp.float32) for _ in range(NBUF)]
        + [pltpu.SemaphoreType.DMA for _ in range(2 * NBUF)]
    ),
)


def kernel(x, table):
    xf = x.reshape(B_TOTAL // CHUNK, CHUNK)
    out = _emb(xf, table)
    return out.reshape(4096, 200, D_MODEL)


# speedup vs baseline: 1.0048x; 1.0048x over previous
import math

import jax
import jax.numpy as jnp
from jax import lax
from jax.experimental import pallas as pl
from jax.experimental.pallas import tpu as pltpu
from jax.experimental.pallas import tpu_sc as plsc

D_MODEL = 64
VOCAB = 1000000
BATCH = 4096
SEQ = 200
SCALE = math.sqrt(D_MODEL)

NC, NS, L = 2, 16, 16
NW = NC * NS
ROWS_PER_W = BATCH // NW
NBUF = 8
OUTER = ROWS_PER_W // NBUF


def _emb_body(x_hbm, table_hbm, out_hbm, idx_v, *scratch):
    rows = scratch[:NBUF]
    gsem = scratch[NBUF:2 * NBUF]
    ssem = scratch[2 * NBUF:3 * NBUF]

    wid = lax.axis_index("s") * NC + lax.axis_index("c")
    row0 = wid * ROWS_PER_W

    pltpu.sync_copy(x_hbm.at[pl.ds(row0, ROWS_PER_W)], idx_v)

    def outer(o, carry):
        for b in range(NBUF):
            i = o * NBUF + b

            @pl.when(o > 0)
            def _drain_prev_scatter(b=b):
                pltpu.make_async_copy(
                    rows[b], out_hbm.at[0], ssem[b]).wait()

            pltpu.async_copy(table_hbm.at[idx_v.at[i]], rows[b], gsem[b])
        for b in range(NBUF):
            i = o * NBUF + b
            pltpu.make_async_copy(
                table_hbm.at[idx_v.at[i]], rows[b], gsem[b]).wait()

            def scale_row(t, c, b=b):
                for j in range(D_MODEL // L):
                    rows[b][t, pl.ds(j * L, L)] = (
                        rows[b][t, pl.ds(j * L, L)] * SCALE)
                return c

            lax.fori_loop(0, SEQ, scale_row, 0)
            pltpu.async_copy(rows[b], out_hbm.at[row0 + i], ssem[b])
        return carry

    lax.fori_loop(0, OUTER, outer, 0)
    for b in range(NBUF):
        pltpu.make_async_copy(rows[b], out_hbm.at[0], ssem[b]).wait()


_emb = pl.kernel(
    _emb_body,
    out_type=jax.ShapeDtypeStruct((BATCH, SEQ, D_MODEL), jnp.float32),
    mesh=plsc.VectorSubcoreMesh(
        core_axis_name="c", subcore_axis_name="s",
        num_cores=NC, num_subcores=NS),
    compiler_params=pltpu.CompilerParams(use_tc_tiling_on_sc=False),
    scratch_types=(
        [pltpu.VMEM((ROWS_PER_W, SEQ), jnp.int32)]
        + [pltpu.VMEM((SEQ, D_MODEL), jnp.float32) for _ in range(NBUF)]
        + [pltpu.SemaphoreType.DMA for _ in range(2 * NBUF)]
    ),
)


def kernel(x, table):
    return _emb(x, table)
